# Initial kernel scaffold; baseline (speedup 1.0000x reference)
import jax
import jax.numpy as jnp
from jax.experimental import pallas as pl


def kernel(x, x_s, edge_index, edge_attr, batch, W1_0, b1_0, W2_0, b2_0, W1_1, b1_1, W2_1, b2_1, W1_2, b1_2, W2_2, b2_2):
    raise NotImplementedError("write your pallas kernel here")


# speedup vs baseline: 2.7890x; 2.7890x over previous
import functools

import jax
import jax.numpy as jnp
from jax import lax
from jax.experimental import pallas as pl
from jax.experimental.pallas import tpu as pltpu
from jax.experimental.pallas import tpu_sc as plsc

N = 10000
E = 320000
D = 128
G = 512
NC = 2
NS = 16
LANES = 16

_MESH = plsc.VectorSubcoreMesh(core_axis_name="c", subcore_axis_name="s")


_E_PER_SUB = E // NS
_CH = 80
_ZROWS = 40
_NROWCH = N // _ZROWS


def _seg_sum_stacked(h2, src, dst):
  @functools.partial(
      pl.kernel,
      mesh=_MESH,
      out_type=jax.ShapeDtypeStruct((2 * N, D), jnp.float32),
      scratch_types=[
          pltpu.VMEM((_CH,), jnp.int32),
          pltpu.VMEM((_CH,), jnp.int32),
          pltpu.VMEM((_CH, D), jnp.float32),
          pltpu.VMEM((_ZROWS, D), jnp.float32),
          pltpu.VMEM_SHARED((N, D), jnp.float32),
          pltpu.SemaphoreType.DMA,
      ],
  )
  def k(h_hbm, src_hbm, dst_hbm, out_hbm, sidx, didx, rows, zbuf, acc, sem):
    cid = lax.axis_index("c")
    sid = lax.axis_index("s")

    @pl.loop(0, _ZROWS)
    def _(r):
      @pl.loop(0, D, step=LANES)
      def _(c0):
        zbuf[r, pl.ds(c0, LANES)] = jnp.zeros((LANES,), jnp.float32)

    @pl.loop(sid, _NROWCH, step=NS)
    def _(rc):
      pltpu.sync_copy(zbuf, acc.at[pl.ds(rc * _ZROWS, _ZROWS)])

    plsc.subcore_barrier()

    row_off = cid * N
    base_e = sid * _E_PER_SUB

    @pl.loop(0, _E_PER_SUB, step=_CH)
    def _(e0):
      pltpu.sync_copy(src_hbm.at[pl.ds(base_e + e0, _CH)], sidx)
      pltpu.sync_copy(dst_hbm.at[pl.ds(base_e + e0, _CH)], didx)

      @pl.loop(0, _CH, step=LANES)
      def _(j):
        sidx[pl.ds(j, LANES)] = sidx[pl.ds(j, LANES)] + row_off

      pltpu.async_copy(h_hbm.at[sidx], rows, sem).wait()
      pltpu.sync_copy(rows, acc.at[didx], add=True)

    plsc.subcore_barrier()

    @pl.loop(sid, _NROWCH, step=NS)
    def _(rc):
      pltpu.sync_copy(
          acc.at[pl.ds(rc * _ZROWS, _ZROWS)],
          out_hbm.at[pl.ds(cid * N + rc * _ZROWS, _ZROWS)])

  return k(h2, src, dst)


_BN = 1000


def _mlp(h2, agg2, W1, b1, W2, b2):
  M = h2.shape[0]

  def body(h_ref, a_ref, w1_ref, b1_ref, w2_ref, b2_ref, o_ref):
    m = h_ref[...] + a_ref[...]
    z = jnp.dot(m, w1_ref[...], preferred_element_type=jnp.float32)
    z = jnp.maximum(z + b1_ref[...], 0.0)
    o = jnp.dot(z, w2_ref[...], preferred_element_type=jnp.float32)
    o_ref[...] = jnp.maximum(o + b2_ref[...], 0.0)

  return pl.pallas_call(
      body,
      grid=(M // _BN,),
      in_specs=[
          pl.BlockSpec((_BN, D), lambda i: (i, 0)),
          pl.BlockSpec((_BN, D), lambda i: (i, 0)),
          pl.BlockSpec((D, D), lambda i: (0, 0)),
          pl.BlockSpec((1, D), lambda i: (0, 0)),
          pl.BlockSpec((D, D), lambda i: (0, 0)),
          pl.BlockSpec((1, D), lambda i: (0, 0)),
      ],
      out_specs=pl.BlockSpec((_BN, D), lambda i: (i, 0)),
      out_shape=jax.ShapeDtypeStruct((M, D), jnp.float32),
  )(h2, agg2, W1, b1.reshape(1, D), W2, b2.reshape(1, D))


_PCH = 40
_NCHUNKS = N // _PCH
_G_PER_SUB = G // NS


def _pool3(h1, h2, h3, batch):
  @functools.partial(
      pl.kernel,
      mesh=_MESH,
      out_type=jax.ShapeDtypeStruct((2, 3, G, D), jnp.float32),
      scratch_types=[
          pltpu.VMEM((_PCH, D), jnp.float32),
          pltpu.VMEM((_PCH,), jnp.int32),
          pltpu.VMEM((_G_PER_SUB, D), jnp.float32),
          pltpu.VMEM_SHARED((G, D), jnp.float32),
          pltpu.VMEM_SHARED((G, D), jnp.float32),
          pltpu.VMEM_SHARED((G, D), jnp.float32),
      ],
  )
  def k(h1_hbm, h2_hbm, h3_hbm, b_hbm, out_hbm, vbuf, bidx, zbuf, p1, p2, p3):
    cid = lax.axis_index("c")
    sid = lax.axis_index("s")

    @pl.loop(0, _G_PER_SUB)
    def _(r):
      @pl.loop(0, D, step=LANES)
      def _(c0):
        zbuf[r, pl.ds(c0, LANES)] = jnp.zeros((LANES,), jnp.float32)

    for p in (p1, p2, p3):
      pltpu.sync_copy(zbuf, p.at[pl.ds(sid * _G_PER_SUB, _G_PER_SUB)])

    plsc.subcore_barrier()

    @pl.loop(sid, _NCHUNKS, step=NS)
    def _(c):
      row0 = c * _PCH
      pltpu.sync_copy(b_hbm.at[pl.ds(row0, _PCH)], bidx)
      for h_hbm, p in ((h1_hbm, p1), (h2_hbm, p2), (h3_hbm, p3)):
        pltpu.sync_copy(h_hbm.at[pl.ds(cid * N + row0, _PCH)], vbuf)
        pltpu.sync_copy(vbuf, p.at[bidx], add=True)

    plsc.subcore_barrier()
    for li, p in enumerate((p1, p2, p3)):
      pltpu.sync_copy(
          p.at[pl.ds(sid * _G_PER_SUB, _G_PER_SUB)],
          out_hbm.at[cid, li, pl.ds(sid * _G_PER_SUB, _G_PER_SUB)])

  return k(h1, h2, h3, batch)


def kernel(x, x_s, edge_index, edge_attr, batch, W1_0, b1_0, W2_0, b2_0,
           W1_1, b1_1, W2_1, b2_1, W1_2, b1_2, W2_2, b2_2):
  del edge_attr
  src = edge_index[0]
  dst = edge_index[1]
  params = ((W1_0, b1_0, W2_0, b2_0),
            (W1_1, b1_1, W2_1, b2_1),
            (W1_2, b1_2, W2_2, b2_2))

  h = jnp.concatenate([x, x_s], axis=0)
  hs = []
  for (W1, b1, W2, b2) in params:
    agg = _seg_sum_stacked(h, src, dst)
    h = _mlp(h, agg, W1, b1, W2, b2)
    hs.append(h)

  pools = _pool3(hs[0], hs[1], hs[2], batch)

  n_x = jnp.concatenate([pools[0, 0], pools[0, 1], pools[0, 2]], axis=1)
  n_xs = jnp.concatenate([pools[1, 0], pools[1, 1], pools[1, 2]], axis=1)
  g_x = jnp.concatenate([hs[0][:N], hs[1][:N], hs[2][:N]], axis=1)
  g_xs = jnp.concatenate([hs[0][N:], hs[1][N:], hs[2][N:]], axis=1)
  return (n_x, g_x, n_xs, g_xs)


# speedup vs baseline: 6.6332x; 2.3783x over previous
import functools

import jax
import jax.numpy as jnp
from jax import lax
from jax.experimental import pallas as pl
from jax.experimental.pallas import tpu as pltpu
from jax.experimental.pallas import tpu_sc as plsc

N = 10000
E = 320000
D = 128
G = 512
NC = 2
NS = 16
LANES = 16

_MESH = plsc.VectorSubcoreMesh(core_axis_name="c", subcore_axis_name="s")


_E_PER_SUB = E // NS
_CH = 80
_ZROWS = 40
_NROWCH = N // _ZROWS


_NECH = _E_PER_SUB // _CH
_EPAD = 2 * _CH


def _seg_sum_stacked(h2, src_p, dst_p):
  idx_scr = []
  for _ in range(4):
    idx_scr += [pltpu.VMEM((_CH,), jnp.int32),
                pltpu.VMEM((_CH,), jnp.int32),
                pltpu.SemaphoreType.DMA]

  @functools.partial(
      pl.kernel,
      mesh=_MESH,
      out_type=jax.ShapeDtypeStruct((2 * N, D), jnp.float32),
      scratch_types=idx_scr + [
          pltpu.VMEM((_CH, D), jnp.float32),
          pltpu.VMEM((_CH, D), jnp.float32),
          pltpu.VMEM((_ZROWS, D), jnp.float32),
          pltpu.VMEM_SHARED((N, D), jnp.float32),
          pltpu.SemaphoreType.DMA,
          pltpu.SemaphoreType.DMA,
      ],
  )
  def k(h_hbm, src_hbm, dst_hbm, out_hbm, *refs):
    (s0, d0, i0, s1, d1, i1, s2, d2, i2, s3, d3, i3,
     bufa, bufb, zbuf, acc, sga, sgb) = refs
    sset = (s0, s1, s2, s3)
    dset = (d0, d1, d2, d3)
    isem = (i0, i1, i2, i3)
    gbuf = (bufa, bufb)
    gsem = (sga, sgb)

    cid = lax.axis_index("c")
    sid = lax.axis_index("s")
    row_off = cid * N
    base_e = sid * _E_PER_SUB

    @pl.loop(0, _ZROWS)
    def _(r):
      @pl.loop(0, D, step=LANES)
      def _(c0):
        zbuf[r, pl.ds(c0, LANES)] = jnp.zeros((LANES,), jnp.float32)

    @pl.loop(sid, _NROWCH, step=NS)
    def _(rc):
      pltpu.sync_copy(zbuf, acc.at[pl.ds(rc * _ZROWS, _ZROWS)])

    plsc.subcore_barrier()

    def i_copies(c, j):
      off = base_e + c * _CH
      return (pltpu.make_async_copy(src_hbm.at[pl.ds(off, _CH)], sset[j],
                                    isem[j]),
              pltpu.make_async_copy(dst_hbm.at[pl.ds(off, _CH)], dset[j],
                                    isem[j]))

    def i_start(c, j):
      for cp in i_copies(c, j):
        cp.start()

    def i_wait(c, j):
      for cp in i_copies(c, j):
        cp.wait()

    def addoff(j):
      @pl.loop(0, _CH, step=LANES)
      def _(t):
        sset[j][pl.ds(t, LANES)] = sset[j][pl.ds(t, LANES)] + row_off

    def gcopy(j, b):
      return pltpu.make_async_copy(h_hbm.at[sset[j]], gbuf[b], gsem[b])

    def put(j, b):
      pltpu.sync_copy(gbuf[b], acc.at[dset[j]], add=True)

    for j in range(4):
      i_start(j, j)
    for j in range(2):
      i_wait(j, j)
      addoff(j)
      gcopy(j, j).start()

    @pl.loop(0, _NECH - 6, step=4)
    def _(c):
      for j in range(4):
        b = j % 2
        gcopy(j, b).wait()
        put(j, b)
        i_start(c + j + 4, j)
        j2 = (j + 2) % 4
        i_wait(c + j + 2, j2)
        addoff(j2)
        gcopy(j2, b).start()

    for cc in range(_NECH - 6, _NECH):
      j = cc % 4
      b = j % 2
      gcopy(j, b).wait()
      put(j, b)
      if cc + 4 < _NECH + 2:
        i_start(cc + 4, j)
      if cc + 2 < _NECH:
        j2 = (j + 2) % 4
        i_wait(cc + 2, j2)
        addoff(j2)
        gcopy(j2, b).start()
      else:
        j2 = (j + 2) % 4
        i_wait(cc + 2, j2)

    plsc.subcore_barrier()

    @pl.loop(sid, _NROWCH, step=NS)
    def _(rc):
      pltpu.sync_copy(
          acc.at[pl.ds(rc * _ZROWS, _ZROWS)],
          out_hbm.at[pl.ds(cid * N + rc * _ZROWS, _ZROWS)])

  return k(h2, src_p, dst_p)


_BN = 1000


def _mlp(h2, agg2, W1, b1, W2, b2):
  M = h2.shape[0]

  def body(h_ref, a_ref, w1_ref, b1_ref, w2_ref, b2_ref, o_ref):
    m = h_ref[...] + a_ref[...]
    z = jnp.dot(m, w1_ref[...], preferred_element_type=jnp.float32)
    z = jnp.maximum(z + b1_ref[...], 0.0)
    o = jnp.dot(z, w2_ref[...], preferred_element_type=jnp.float32)
    o_ref[...] = jnp.maximum(o + b2_ref[...], 0.0)

  return pl.pallas_call(
      body,
      grid=(M // _BN,),
      in_specs=[
          pl.BlockSpec((_BN, D), lambda i: (i, 0)),
          pl.BlockSpec((_BN, D), lambda i: (i, 0)),
          pl.BlockSpec((D, D), lambda i: (0, 0)),
          pl.BlockSpec((1, D), lambda i: (0, 0)),
          pl.BlockSpec((D, D), lambda i: (0, 0)),
          pl.BlockSpec((1, D), lambda i: (0, 0)),
      ],
      out_specs=pl.BlockSpec((_BN, D), lambda i: (i, 0)),
      out_shape=jax.ShapeDtypeStruct((M, D), jnp.float32),
  )(h2, agg2, W1, b1.reshape(1, D), W2, b2.reshape(1, D))


_PCH = 40
_NCHUNKS = N // _PCH
_G_PER_SUB = G // NS


def _pool3(h1, h2, h3, batch):
  @functools.partial(
      pl.kernel,
      mesh=_MESH,
      out_type=jax.ShapeDtypeStruct((2, 3, G, D), jnp.float32),
      scratch_types=[
          pltpu.VMEM((_PCH, D), jnp.float32),
          pltpu.VMEM((_PCH,), jnp.int32),
          pltpu.VMEM((_G_PER_SUB, D), jnp.float32),
          pltpu.VMEM_SHARED((G, D), jnp.float32),
          pltpu.VMEM_SHARED((G, D), jnp.float32),
          pltpu.VMEM_SHARED((G, D), jnp.float32),
      ],
  )
  def k(h1_hbm, h2_hbm, h3_hbm, b_hbm, out_hbm, vbuf, bidx, zbuf, p1, p2, p3):
    cid = lax.axis_index("c")
    sid = lax.axis_index("s")

    @pl.loop(0, _G_PER_SUB)
    def _(r):
      @pl.loop(0, D, step=LANES)
      def _(c0):
        zbuf[r, pl.ds(c0, LANES)] = jnp.zeros((LANES,), jnp.float32)

    for p in (p1, p2, p3):
      pltpu.sync_copy(zbuf, p.at[pl.ds(sid * _G_PER_SUB, _G_PER_SUB)])

    plsc.subcore_barrier()

    @pl.loop(sid, _NCHUNKS, step=NS)
    def _(c):
      row0 = c * _PCH
      pltpu.sync_copy(b_hbm.at[pl.ds(row0, _PCH)], bidx)
      for h_hbm, p in ((h1_hbm, p1), (h2_hbm, p2), (h3_hbm, p3)):
        pltpu.sync_copy(h_hbm.at[pl.ds(cid * N + row0, _PCH)], vbuf)
        pltpu.sync_copy(vbuf, p.at[bidx], add=True)

    plsc.subcore_barrier()
    for li, p in enumerate((p1, p2, p3)):
      pltpu.sync_copy(
          p.at[pl.ds(sid * _G_PER_SUB, _G_PER_SUB)],
          out_hbm.at[cid, li, pl.ds(sid * _G_PER_SUB, _G_PER_SUB)])

  return k(h1, h2, h3, batch)


def kernel(x, x_s, edge_index, edge_attr, batch, W1_0, b1_0, W2_0, b2_0,
           W1_1, b1_1, W2_1, b2_1, W1_2, b1_2, W2_2, b2_2):
  del edge_attr
  pad = jnp.zeros((_EPAD,), jnp.int32)
  src_p = jnp.concatenate([edge_index[0], pad])
  dst_p = jnp.concatenate([edge_index[1], pad])
  params = ((W1_0, b1_0, W2_0, b2_0),
            (W1_1, b1_1, W2_1, b2_1),
            (W1_2, b1_2, W2_2, b2_2))

  h = jnp.concatenate([x, x_s], axis=0)
  hs = []
  for (W1, b1, W2, b2) in params:
    agg = _seg_sum_stacked(h, src_p, dst_p)
    h = _mlp(h, agg, W1, b1, W2, b2)
    hs.append(h)

  pools = _pool3(hs[0], hs[1], hs[2], batch)

  n_x = jnp.concatenate([pools[0, 0], pools[0, 1], pools[0, 2]], axis=1)
  n_xs = jnp.concatenate([pools[1, 0], pools[1, 1], pools[1, 2]], axis=1)
  g_x = jnp.concatenate([hs[0][:N], hs[1][:N], hs[2][:N]], axis=1)
  g_xs = jnp.concatenate([hs[0][N:], hs[1][N:], hs[2][N:]], axis=1)
  return (n_x, g_x, n_xs, g_xs)


# speedup vs baseline: 7.3559x; 1.1089x over previous
import functools

import jax
import jax.numpy as jnp
from jax import lax
from jax.experimental import pallas as pl
from jax.experimental.pallas import tpu as pltpu
from jax.experimental.pallas import tpu_sc as plsc

N = 10000
E = 320000
D = 128
G = 512
NC = 2
NS = 16
LANES = 16

_MESH = plsc.VectorSubcoreMesh(core_axis_name="c", subcore_axis_name="s")


_E_PER_SUB = E // NS
_CH = 128
_TAIL = _E_PER_SUB % _CH
_ZROWS = 40
_NROWCH = N // _ZROWS


_NECH = _E_PER_SUB // _CH
_EPAD = 2 * _CH


def _seg_sum_stacked(h2, src_p, dst_p):
  idx_scr = []
  for _ in range(4):
    idx_scr += [pltpu.VMEM((_CH,), jnp.int32),
                pltpu.VMEM((_CH,), jnp.int32),
                pltpu.SemaphoreType.DMA]

  @functools.partial(
      pl.kernel,
      mesh=_MESH,
      out_type=jax.ShapeDtypeStruct((2 * N, D), jnp.float32),
      scratch_types=idx_scr + [
          pltpu.VMEM((_CH, D), jnp.float32),
          pltpu.VMEM((_CH, D), jnp.float32),
          pltpu.VMEM((_TAIL,), jnp.int32),
          pltpu.VMEM((_TAIL,), jnp.int32),
          pltpu.VMEM((_TAIL, D), jnp.float32),
          pltpu.VMEM((_ZROWS, D), jnp.float32),
          pltpu.VMEM_SHARED((N, D), jnp.float32),
          pltpu.SemaphoreType.DMA,
          pltpu.SemaphoreType.DMA,
          pltpu.SemaphoreType.DMA,
      ],
  )
  def k(h_hbm, src_hbm, dst_hbm, out_hbm, *refs):
    (s0, d0, i0, s1, d1, i1, s2, d2, i2, s3, d3, i3,
     bufa, bufb, stl, dtl, buft, zbuf, acc, sga, sgb, szw) = refs
    sset = (s0, s1, s2, s3)
    dset = (d0, d1, d2, d3)
    isem = (i0, i1, i2, i3)
    gbuf = (bufa, bufb)
    gsem = (sga, sgb)

    cid = lax.axis_index("c")
    sid = lax.axis_index("s")
    row_off = cid * N
    base_e = sid * _E_PER_SUB

    @pl.loop(0, _ZROWS)
    def _(r):
      @pl.loop(0, D, step=LANES)
      def _(c0):
        zbuf[r, pl.ds(c0, LANES)] = jnp.zeros((LANES,), jnp.float32)

    def zcopy(rc):
      return pltpu.make_async_copy(zbuf, acc.at[pl.ds(rc * _ZROWS, _ZROWS)],
                                   szw)

    @pl.loop(sid, _NROWCH, step=NS)
    def _(rc):
      zcopy(rc).start()

    @pl.loop(sid, _NROWCH, step=NS)
    def _(rc):
      zcopy(rc).wait()

    plsc.subcore_barrier()

    def i_copies(c, j):
      off = base_e + c * _CH
      return (pltpu.make_async_copy(src_hbm.at[pl.ds(off, _CH)], sset[j],
                                    isem[j]),
              pltpu.make_async_copy(dst_hbm.at[pl.ds(off, _CH)], dset[j],
                                    isem[j]))

    def i_start(c, j):
      for cp in i_copies(c, j):
        cp.start()

    def i_wait(c, j):
      for cp in i_copies(c, j):
        cp.wait()

    def addoff(j):
      @pl.loop(0, _CH, step=LANES)
      def _(t):
        sset[j][pl.ds(t, LANES)] = sset[j][pl.ds(t, LANES)] + row_off

    def gcopy(j, b):
      return pltpu.make_async_copy(h_hbm.at[sset[j]], gbuf[b], gsem[b])

    def put(j, b):
      pltpu.sync_copy(gbuf[b], acc.at[dset[j]], add=True)

    for j in range(4):
      i_start(j, j)
    for j in range(2):
      i_wait(j, j)
      addoff(j)
      gcopy(j, j).start()

    @pl.loop(0, _NECH - 8, step=4)
    def _(c):
      for j in range(4):
        b = j % 2
        gcopy(j, b).wait()
        put(j, b)
        i_start(c + j + 4, j)
        j2 = (j + 2) % 4
        i_wait(c + j + 2, j2)
        addoff(j2)
        gcopy(j2, b).start()

    for cc in range(_NECH - 8, _NECH):
      j = cc % 4
      b = j % 2
      gcopy(j, b).wait()
      put(j, b)
      if cc + 4 < _NECH + 2:
        i_start(cc + 4, j)
      j2 = (j + 2) % 4
      if cc + 2 < _NECH:
        i_wait(cc + 2, j2)
        addoff(j2)
        gcopy(j2, b).start()
      else:
        i_wait(cc + 2, j2)

    toff = base_e + _NECH * _CH
    pltpu.sync_copy(src_hbm.at[pl.ds(toff, _TAIL)], stl)
    pltpu.sync_copy(dst_hbm.at[pl.ds(toff, _TAIL)], dtl)

    @pl.loop(0, _TAIL, step=LANES)
    def _(t):
      stl[pl.ds(t, LANES)] = stl[pl.ds(t, LANES)] + row_off

    pltpu.async_copy(h_hbm.at[stl], buft, sga).wait()
    pltpu.sync_copy(buft, acc.at[dtl], add=True)

    plsc.subcore_barrier()

    def wcopy(rc):
      return pltpu.make_async_copy(
          acc.at[pl.ds(rc * _ZROWS, _ZROWS)],
          out_hbm.at[pl.ds(cid * N + rc * _ZROWS, _ZROWS)], szw)

    @pl.loop(sid, _NROWCH, step=NS)
    def _(rc):
      wcopy(rc).start()

    @pl.loop(sid, _NROWCH, step=NS)
    def _(rc):
      wcopy(rc).wait()

  return k(h2, src_p, dst_p)


_BN = 1000


def _mlp(h2, agg2, W1, b1, W2, b2):
  M = h2.shape[0]

  def body(h_ref, a_ref, w1_ref, b1_ref, w2_ref, b2_ref, o_ref):
    m = h_ref[...] + a_ref[...]
    z = jnp.dot(m, w1_ref[...], preferred_element_type=jnp.float32)
    z = jnp.maximum(z + b1_ref[...], 0.0)
    o = jnp.dot(z, w2_ref[...], preferred_element_type=jnp.float32)
    o_ref[...] = jnp.maximum(o + b2_ref[...], 0.0)

  return pl.pallas_call(
      body,
      grid=(M // _BN,),
      in_specs=[
          pl.BlockSpec((_BN, D), lambda i: (i, 0)),
          pl.BlockSpec((_BN, D), lambda i: (i, 0)),
          pl.BlockSpec((D, D), lambda i: (0, 0)),
          pl.BlockSpec((1, D), lambda i: (0, 0)),
          pl.BlockSpec((D, D), lambda i: (0, 0)),
          pl.BlockSpec((1, D), lambda i: (0, 0)),
      ],
      out_specs=pl.BlockSpec((_BN, D), lambda i: (i, 0)),
      out_shape=jax.ShapeDtypeStruct((M, D), jnp.float32),
  )(h2, agg2, W1, b1.reshape(1, D), W2, b2.reshape(1, D))


_PCH = 40
_NCHUNKS = N // _PCH
_G_PER_SUB = G // NS


def _pool3(h1, h2, h3, batch):
  @functools.partial(
      pl.kernel,
      mesh=_MESH,
      out_type=jax.ShapeDtypeStruct((2, 3, G, D), jnp.float32),
      scratch_types=[
          pltpu.VMEM((_PCH, D), jnp.float32),
          pltpu.VMEM((_PCH,), jnp.int32),
          pltpu.VMEM((_G_PER_SUB, D), jnp.float32),
          pltpu.VMEM_SHARED((G, D), jnp.float32),
          pltpu.VMEM_SHARED((G, D), jnp.float32),
          pltpu.VMEM_SHARED((G, D), jnp.float32),
      ],
  )
  def k(h1_hbm, h2_hbm, h3_hbm, b_hbm, out_hbm, vbuf, bidx, zbuf, p1, p2, p3):
    cid = lax.axis_index("c")
    sid = lax.axis_index("s")

    @pl.loop(0, _G_PER_SUB)
    def _(r):
      @pl.loop(0, D, step=LANES)
      def _(c0):
        zbuf[r, pl.ds(c0, LANES)] = jnp.zeros((LANES,), jnp.float32)

    for p in (p1, p2, p3):
      pltpu.sync_copy(zbuf, p.at[pl.ds(sid * _G_PER_SUB, _G_PER_SUB)])

    plsc.subcore_barrier()

    @pl.loop(sid, _NCHUNKS, step=NS)
    def _(c):
      row0 = c * _PCH
      pltpu.sync_copy(b_hbm.at[pl.ds(row0, _PCH)], bidx)
      for h_hbm, p in ((h1_hbm, p1), (h2_hbm, p2), (h3_hbm, p3)):
        pltpu.sync_copy(h_hbm.at[pl.ds(cid * N + row0, _PCH)], vbuf)
        pltpu.sync_copy(vbuf, p.at[bidx], add=True)

    plsc.subcore_barrier()
    for li, p in enumerate((p1, p2, p3)):
      pltpu.sync_copy(
          p.at[pl.ds(sid * _G_PER_SUB, _G_PER_SUB)],
          out_hbm.at[cid, li, pl.ds(sid * _G_PER_SUB, _G_PER_SUB)])

  return k(h1, h2, h3, batch)


def kernel(x, x_s, edge_index, edge_attr, batch, W1_0, b1_0, W2_0, b2_0,
           W1_1, b1_1, W2_1, b2_1, W1_2, b1_2, W2_2, b2_2):
  del edge_attr
  pad = jnp.zeros((_EPAD,), jnp.int32)
  src_p = jnp.concatenate([edge_index[0], pad])
  dst_p = jnp.concatenate([edge_index[1], pad])
  params = ((W1_0, b1_0, W2_0, b2_0),
            (W1_1, b1_1, W2_1, b2_1),
            (W1_2, b1_2, W2_2, b2_2))

  h = jnp.concatenate([x, x_s], axis=0)
  hs = []
  for (W1, b1, W2, b2) in params:
    agg = _seg_sum_stacked(h, src_p, dst_p)
    h = _mlp(h, agg, W1, b1, W2, b2)
    hs.append(h)

  pools = _pool3(hs[0], hs[1], hs[2], batch)

  n_x = jnp.concatenate([pools[0, 0], pools[0, 1], pools[0, 2]], axis=1)
  n_xs = jnp.concatenate([pools[1, 0], pools[1, 1], pools[1, 2]], axis=1)
  g_x = jnp.concatenate([hs[0][:N], hs[1][:N], hs[2][:N]], axis=1)
  g_xs = jnp.concatenate([hs[0][N:], hs[1][N:], hs[2][N:]], axis=1)
  return (n_x, g_x, n_xs, g_xs)


# speedup vs baseline: 7.3645x; 1.0012x over previous
import functools

import jax
import jax.numpy as jnp
from jax import lax
from jax.experimental import pallas as pl
from jax.experimental.pallas import tpu as pltpu
from jax.experimental.pallas import tpu_sc as plsc

N = 10000
E = 320000
D = 128
G = 512
NC = 2
NS = 16
LANES = 16

_MESH = plsc.VectorSubcoreMesh(core_axis_name="c", subcore_axis_name="s")


_E_PER_SUB = E // NS
_CH = 128
_TAIL = _E_PER_SUB % _CH
_ZROWS = 40
_NROWCH = N // _ZROWS


_NECH = _E_PER_SUB // _CH
_EPAD = 2 * _CH


def _seg_sum_stacked(h2, src_p, dst_p):
  idx_scr = []
  for _ in range(4):
    idx_scr += [pltpu.VMEM((_CH,), jnp.int32),
                pltpu.VMEM((_CH,), jnp.int32),
                pltpu.SemaphoreType.DMA]

  @functools.partial(
      pl.kernel,
      mesh=_MESH,
      out_type=jax.ShapeDtypeStruct((2 * N, D), jnp.float32),
      scratch_types=idx_scr + [
          pltpu.VMEM((_CH, D), jnp.float32),
          pltpu.VMEM((_CH, D), jnp.float32),
          pltpu.VMEM((_TAIL,), jnp.int32),
          pltpu.VMEM((_TAIL,), jnp.int32),
          pltpu.VMEM((_TAIL, D), jnp.float32),
          pltpu.VMEM((_ZROWS, D), jnp.float32),
          pltpu.VMEM_SHARED((N, D), jnp.float32),
          pltpu.SemaphoreType.DMA,
          pltpu.SemaphoreType.DMA,
          pltpu.SemaphoreType.DMA,
      ],
  )
  def k(h_hbm, src_hbm, dst_hbm, out_hbm, *refs):
    (s0, d0, i0, s1, d1, i1, s2, d2, i2, s3, d3, i3,
     bufa, bufb, stl, dtl, buft, zbuf, acc, sga, sgb, szw) = refs
    sset = (s0, s1, s2, s3)
    dset = (d0, d1, d2, d3)
    isem = (i0, i1, i2, i3)
    gbuf = (bufa, bufb)
    gsem = (sga, sgb)

    cid = lax.axis_index("c")
    sid = lax.axis_index("s")
    row_off = cid * N
    base_e = sid * _E_PER_SUB

    @pl.loop(0, _ZROWS)
    def _(r):
      @pl.loop(0, D, step=LANES)
      def _(c0):
        zbuf[r, pl.ds(c0, LANES)] = jnp.zeros((LANES,), jnp.float32)

    def zcopy(rc):
      return pltpu.make_async_copy(zbuf, acc.at[pl.ds(rc * _ZROWS, _ZROWS)],
                                   szw)

    @pl.loop(sid, _NROWCH, step=NS)
    def _(rc):
      zcopy(rc).start()

    @pl.loop(sid, _NROWCH, step=NS)
    def _(rc):
      zcopy(rc).wait()

    plsc.subcore_barrier()

    def i_copies(c, j):
      off = base_e + c * _CH
      return (pltpu.make_async_copy(src_hbm.at[pl.ds(off, _CH)], sset[j],
                                    isem[j]),
              pltpu.make_async_copy(dst_hbm.at[pl.ds(off, _CH)], dset[j],
                                    isem[j]))

    def i_start(c, j):
      for cp in i_copies(c, j):
        cp.start()

    def i_wait(c, j):
      for cp in i_copies(c, j):
        cp.wait()

    def addoff(j):
      @pl.loop(0, _CH, step=LANES)
      def _(t):
        sset[j][pl.ds(t, LANES)] = sset[j][pl.ds(t, LANES)] + row_off

    def gcopy(j, b):
      return pltpu.make_async_copy(h_hbm.at[sset[j]], gbuf[b], gsem[b])

    def put(j, b):
      pltpu.sync_copy(gbuf[b], acc.at[dset[j]], add=True)

    for j in range(4):
      i_start(j, j)
    for j in range(2):
      i_wait(j, j)
      addoff(j)
      gcopy(j, j).start()

    @pl.loop(0, _NECH - 8, step=4)
    def _(c):
      for j in range(4):
        b = j % 2
        gcopy(j, b).wait()
        put(j, b)
        i_start(c + j + 4, j)
        j2 = (j + 2) % 4
        i_wait(c + j + 2, j2)
        addoff(j2)
        gcopy(j2, b).start()

    for cc in range(_NECH - 8, _NECH):
      j = cc % 4
      b = j % 2
      gcopy(j, b).wait()
      put(j, b)
      if cc + 4 < _NECH + 2:
        i_start(cc + 4, j)
      j2 = (j + 2) % 4
      if cc + 2 < _NECH:
        i_wait(cc + 2, j2)
        addoff(j2)
        gcopy(j2, b).start()
      else:
        i_wait(cc + 2, j2)

    toff = base_e + _NECH * _CH
    pltpu.sync_copy(src_hbm.at[pl.ds(toff, _TAIL)], stl)
    pltpu.sync_copy(dst_hbm.at[pl.ds(toff, _TAIL)], dtl)

    @pl.loop(0, _TAIL, step=LANES)
    def _(t):
      stl[pl.ds(t, LANES)] = stl[pl.ds(t, LANES)] + row_off

    pltpu.async_copy(h_hbm.at[stl], buft, sga).wait()
    pltpu.sync_copy(buft, acc.at[dtl], add=True)

    plsc.subcore_barrier()

    def wcopy(rc):
      return pltpu.make_async_copy(
          acc.at[pl.ds(rc * _ZROWS, _ZROWS)],
          out_hbm.at[pl.ds(cid * N + rc * _ZROWS, _ZROWS)], szw)

    @pl.loop(sid, _NROWCH, step=NS)
    def _(rc):
      wcopy(rc).start()

    @pl.loop(sid, _NROWCH, step=NS)
    def _(rc):
      wcopy(rc).wait()

  return k(h2, src_p, dst_p)


_BN = 1000


def _mlp(h2, agg2, W1, b1, W2, b2):
  M = h2.shape[0]

  def body(h_ref, a_ref, w1_ref, b1_ref, w2_ref, b2_ref, o_ref):
    m = h_ref[...] + a_ref[...]
    z = jnp.dot(m, w1_ref[...], preferred_element_type=jnp.float32)
    z = jnp.maximum(z + b1_ref[...], 0.0)
    o = jnp.dot(z, w2_ref[...], preferred_element_type=jnp.float32)
    o_ref[...] = jnp.maximum(o + b2_ref[...], 0.0)

  return pl.pallas_call(
      body,
      grid=(M // _BN,),
      in_specs=[
          pl.BlockSpec((_BN, D), lambda i: (i, 0)),
          pl.BlockSpec((_BN, D), lambda i: (i, 0)),
          pl.BlockSpec((D, D), lambda i: (0, 0)),
          pl.BlockSpec((1, D), lambda i: (0, 0)),
          pl.BlockSpec((D, D), lambda i: (0, 0)),
          pl.BlockSpec((1, D), lambda i: (0, 0)),
      ],
      out_specs=pl.BlockSpec((_BN, D), lambda i: (i, 0)),
      out_shape=jax.ShapeDtypeStruct((M, D), jnp.float32),
  )(h2, agg2, W1, b1.reshape(1, D), W2, b2.reshape(1, D))


_BNP = 1000
_NBP = N // _BNP


def _pool_tc(h2, batch3):
  def body(b_ref, h_ref, o_ref):
    j = pl.program_id(1)

    @pl.when(j == 0)
    def _():
      o_ref[...] = jnp.zeros_like(o_ref)

    bidx = b_ref[0, 0]
    iota = lax.broadcasted_iota(jnp.int32, (G, _BNP), 0)
    onehot = (iota == bidx[None, :]).astype(jnp.float32)
    o_ref[0] += jnp.dot(onehot, h_ref[0], preferred_element_type=jnp.float32,
                        precision=lax.Precision.HIGHEST)

  return pl.pallas_call(
      body,
      grid=(2, _NBP),
      in_specs=[
          pl.BlockSpec((1, 1, _BNP), lambda e, j: (j, 0, 0)),
          pl.BlockSpec((1, _BNP, D), lambda e, j: (e, j, 0)),
      ],
      out_specs=pl.BlockSpec((1, G, D), lambda e, j: (e, 0, 0)),
      out_shape=jax.ShapeDtypeStruct((2, G, D), jnp.float32),
  )(batch3, h2.reshape(2, N, D))


def kernel(x, x_s, edge_index, edge_attr, batch, W1_0, b1_0, W2_0, b2_0,
           W1_1, b1_1, W2_1, b2_1, W1_2, b1_2, W2_2, b2_2):
  del edge_attr
  pad = jnp.zeros((_EPAD,), jnp.int32)
  src_p = jnp.concatenate([edge_index[0], pad])
  dst_p = jnp.concatenate([edge_index[1], pad])
  params = ((W1_0, b1_0, W2_0, b2_0),
            (W1_1, b1_1, W2_1, b2_1),
            (W1_2, b1_2, W2_2, b2_2))

  batch3 = batch.reshape(_NBP, 1, _BNP)
  h = jnp.concatenate([x, x_s], axis=0)
  hs, pools = [], []
  for (W1, b1, W2, b2) in params:
    agg = _seg_sum_stacked(h, src_p, dst_p)
    h = _mlp(h, agg, W1, b1, W2, b2)
    hs.append(h)
    pools.append(_pool_tc(h, batch3))

  n_x = jnp.concatenate([p[0] for p in pools], axis=1)
  n_xs = jnp.concatenate([p[1] for p in pools], axis=1)
  g_x = jnp.concatenate([hs[0][:N], hs[1][:N], hs[2][:N]], axis=1)
  g_xs = jnp.concatenate([hs[0][N:], hs[1][N:], hs[2][N:]], axis=1)
  return (n_x, g_x, n_xs, g_xs)


# speedup vs baseline: 7.6290x; 1.0359x over previous
import functools

import jax
import jax.numpy as jnp
from jax import lax
from jax.experimental import pallas as pl
from jax.experimental.pallas import tpu as pltpu
from jax.experimental.pallas import tpu_sc as plsc

N = 10000
E = 320000
D = 128
G = 512
NC = 2
NS = 16
LANES = 16

_MESH = plsc.VectorSubcoreMesh(core_axis_name="c", subcore_axis_name="s")


_E_PER_SUB = E // NS
_CH = 80
_NECH = _E_PER_SUB // _CH
_EPAD = 2 * _CH
_EP = E + _EPAD
_ZROWS = 40
_NROWCH = N // _ZROWS


def _seg_sum_stacked(h2, src2, dst_p):
  idx_scr = []
  for _ in range(8):
    idx_scr += [pltpu.VMEM((_CH,), jnp.int32),
                pltpu.VMEM((_CH,), jnp.int32),
                pltpu.SemaphoreType.DMA]

  @functools.partial(
      pl.kernel,
      mesh=_MESH,
      out_type=jax.ShapeDtypeStruct((2 * N, D), jnp.float32),
      scratch_types=idx_scr + [
          pltpu.VMEM((_CH, D), jnp.float32),
          pltpu.VMEM((_CH, D), jnp.float32),
          pltpu.VMEM((_CH, D), jnp.float32),
          pltpu.VMEM((_CH, D), jnp.float32),
          pltpu.VMEM((_ZROWS, D), jnp.float32),
          pltpu.VMEM_SHARED((N, D), jnp.float32),
          pltpu.SemaphoreType.DMA,
          pltpu.SemaphoreType.DMA,
          pltpu.SemaphoreType.DMA,
          pltpu.SemaphoreType.DMA,
          pltpu.SemaphoreType.DMA,
          pltpu.SemaphoreType.DMA,
          pltpu.SemaphoreType.DMA,
          pltpu.SemaphoreType.DMA,
          pltpu.SemaphoreType.DMA,
      ],
  )
  def k(h_hbm, src_hbm, dst_hbm, out_hbm, *refs):
    sset = [refs[3 * i] for i in range(8)]
    dset = [refs[3 * i + 1] for i in range(8)]
    isem = [refs[3 * i + 2] for i in range(8)]
    gbuf = list(refs[24:28])
    zbuf = refs[28]
    acc = refs[29]
    gsem = list(refs[30:34])
    ssem = list(refs[34:38])
    szw = refs[38]

    cid = lax.axis_index("c")
    sid = lax.axis_index("s")
    base_e = sid * _E_PER_SUB

    @pl.loop(0, _ZROWS)
    def _(r):
      @pl.loop(0, D, step=LANES)
      def _(c0):
        zbuf[r, pl.ds(c0, LANES)] = jnp.zeros((LANES,), jnp.float32)

    def zcopy(rc):
      return pltpu.make_async_copy(zbuf, acc.at[pl.ds(rc * _ZROWS, _ZROWS)],
                                   szw)

    @pl.loop(sid, _NROWCH, step=NS)
    def _(rc):
      zcopy(rc).start()

    def i_copies(c, j8):
      soff = cid * _EP + base_e + c * _CH
      doff = base_e + c * _CH
      return (pltpu.make_async_copy(src_hbm.at[pl.ds(soff, _CH)], sset[j8],
                                    isem[j8]),
              pltpu.make_async_copy(dst_hbm.at[pl.ds(doff, _CH)], dset[j8],
                                    isem[j8]))

    def i_start(c, j8):
      for cp in i_copies(c, j8):
        cp.start()

    def i_wait(c, j8):
      for cp in i_copies(c, j8):
        cp.wait()

    def gcopy(j8, b4):
      return pltpu.make_async_copy(h_hbm.at[sset[j8]], gbuf[b4], gsem[b4])

    def scopy(j8, b4):
      return pltpu.make_async_copy(gbuf[b4], acc.at[dset[j8]], ssem[b4])

    def slot(c, j8):
      if isinstance(c, int):
        real = lambda x: 0 <= x < _NECH
      else:
        real = lambda x: True
      cm3, cm4, cp4 = c - 3, c - 4, c + 4
      if real(cm3):
        gcopy((j8 + 5) % 8, (j8 + 1) % 4).wait()
        scopy((j8 + 5) % 8, (j8 + 1) % 4).start(add=True)
      if real(cm4):
        scopy((j8 + 4) % 8, j8 % 4).wait()
      if not isinstance(c, int) or cp4 < _NECH + 2:
        i_start(cp4, (j8 + 4) % 8)
      if real(c):
        i_wait(c, j8)
        gcopy(j8, j8 % 4).start()

    for j in range(4):
      i_start(j, j)
    for cc in range(0, 3):
      slot(cc, cc)

    @pl.loop(sid, _NROWCH, step=NS)
    def _(rc):
      zcopy(rc).wait()

    plsc.subcore_barrier()

    for cc in range(3, 8):
      slot(cc, cc)

    @pl.loop(8, 233, step=8)
    def _(c):
      for j in range(8):
        slot(c + j, j)

    for cc in range(240, 250):
      slot(cc, cc % 8)

    for cc in (247, 248, 249):
      gcopy(cc % 8, cc % 4).wait()
      scopy(cc % 8, cc % 4).start(add=True)
    for cc in (246, 247, 248, 249):
      scopy(cc % 8, cc % 4).wait()
    for cc in (250, 251):
      i_wait(cc, cc % 8)

    plsc.subcore_barrier()

    def wcopy(rc):
      return pltpu.make_async_copy(
          acc.at[pl.ds(rc * _ZROWS, _ZROWS)],
          out_hbm.at[pl.ds(cid * N + rc * _ZROWS, _ZROWS)], szw)

    @pl.loop(sid, _NROWCH, step=NS)
    def _(rc):
      wcopy(rc).start()

    @pl.loop(sid, _NROWCH, step=NS)
    def _(rc):
      wcopy(rc).wait()

  return k(h2, src2, dst_p)


_BN = 1000


def _mlp(h2, agg2, W1, b1, W2, b2):
  M = h2.shape[0]

  def body(h_ref, a_ref, w1_ref, b1_ref, w2_ref, b2_ref, o_ref):
    m = h_ref[...] + a_ref[...]
    z = jnp.dot(m, w1_ref[...], preferred_element_type=jnp.float32)
    z = jnp.maximum(z + b1_ref[...], 0.0)
    o = jnp.dot(z, w2_ref[...], preferred_element_type=jnp.float32)
    o_ref[...] = jnp.maximum(o + b2_ref[...], 0.0)

  return pl.pallas_call(
      body,
      grid=(M // _BN,),
      in_specs=[
          pl.BlockSpec((_BN, D), lambda i: (i, 0)),
          pl.BlockSpec((_BN, D), lambda i: (i, 0)),
          pl.BlockSpec((D, D), lambda i: (0, 0)),
          pl.BlockSpec((1, D), lambda i: (0, 0)),
          pl.BlockSpec((D, D), lambda i: (0, 0)),
          pl.BlockSpec((1, D), lambda i: (0, 0)),
      ],
      out_specs=pl.BlockSpec((_BN, D), lambda i: (i, 0)),
      out_shape=jax.ShapeDtypeStruct((M, D), jnp.float32),
  )(h2, agg2, W1, b1.reshape(1, D), W2, b2.reshape(1, D))


_BNP = 1000
_NBP = N // _BNP


def _pool_tc(h2, batch3):
  def body(b_ref, h_ref, o_ref):
    j = pl.program_id(1)

    @pl.when(j == 0)
    def _():
      o_ref[...] = jnp.zeros_like(o_ref)

    bidx = b_ref[0, 0]
    iota = lax.broadcasted_iota(jnp.int32, (G, _BNP), 0)
    onehot = (iota == bidx[None, :]).astype(jnp.float32)
    o_ref[0] += jnp.dot(onehot, h_ref[0], preferred_element_type=jnp.float32,
                        precision=lax.Precision.HIGHEST)

  return pl.pallas_call(
      body,
      grid=(2, _NBP),
      in_specs=[
          pl.BlockSpec((1, 1, _BNP), lambda e, j: (j, 0, 0)),
          pl.BlockSpec((1, _BNP, D), lambda e, j: (e, j, 0)),
      ],
      out_specs=pl.BlockSpec((1, G, D), lambda e, j: (e, 0, 0)),
      out_shape=jax.ShapeDtypeStruct((2, G, D), jnp.float32),
  )(batch3, h2.reshape(2, N, D))


def kernel(x, x_s, edge_index, edge_attr, batch, W1_0, b1_0, W2_0, b2_0,
           W1_1, b1_1, W2_1, b2_1, W1_2, b1_2, W2_2, b2_2):
  del edge_attr
  pad = jnp.zeros((_EPAD,), jnp.int32)
  src_p = jnp.concatenate([edge_index[0], pad])
  src2 = jnp.concatenate([src_p, src_p + N])
  dst_p = jnp.concatenate([edge_index[1], pad])
  params = ((W1_0, b1_0, W2_0, b2_0),
            (W1_1, b1_1, W2_1, b2_1),
            (W1_2, b1_2, W2_2, b2_2))

  batch3 = batch.reshape(_NBP, 1, _BNP)
  h = jnp.concatenate([x, x_s], axis=0)
  hs, pools = [], []
  for (W1, b1, W2, b2) in params:
    agg = _seg_sum_stacked(h, src2, dst_p)
    h = _mlp(h, agg, W1, b1, W2, b2)
    hs.append(h)
    pools.append(_pool_tc(h, batch3))

  n_x = jnp.concatenate([p[0] for p in pools], axis=1)
  n_xs = jnp.concatenate([p[1] for p in pools], axis=1)
  g_x = jnp.concatenate([hs[0][:N], hs[1][:N], hs[2][:N]], axis=1)
  g_xs = jnp.concatenate([hs[0][N:], hs[1][N:], hs[2][N:]], axis=1)
  return (n_x, g_x, n_xs, g_xs)


# speedup vs baseline: 7.8511x; 1.0291x over previous
import functools

import jax
import jax.numpy as jnp
from jax import lax
from jax.experimental import pallas as pl
from jax.experimental.pallas import tpu as pltpu
from jax.experimental.pallas import tpu_sc as plsc

N = 10000
E = 320000
D = 128
G = 512
NC = 2
NS = 16
LANES = 16

_MESH = plsc.VectorSubcoreMesh(core_axis_name="c", subcore_axis_name="s")


_E_PER_SUB = E // NS
_CH = 80
_NECH = _E_PER_SUB // _CH
_EPAD = 2 * _CH
_EP = E + _EPAD
_ZROWS = 40
_NROWCH = N // _ZROWS


def _seg_sum_stacked(h2, src2, dst_p):
  idx_scr = []
  for _ in range(8):
    idx_scr += [pltpu.VMEM((_CH,), jnp.int32),
                pltpu.VMEM((_CH,), jnp.int32),
                pltpu.SemaphoreType.DMA]

  @functools.partial(
      pl.kernel,
      mesh=_MESH,
      out_type=jax.ShapeDtypeStruct((2 * N, D), jnp.float32),
      scratch_types=idx_scr + [
          pltpu.VMEM((_CH, D), jnp.float32),
          pltpu.VMEM((_CH, D), jnp.float32),
          pltpu.VMEM((_CH, D), jnp.float32),
          pltpu.VMEM((_CH, D), jnp.float32),
          pltpu.VMEM((_ZROWS, D), jnp.float32),
          pltpu.VMEM_SHARED((N, D), jnp.float32),
          pltpu.SemaphoreType.DMA,
          pltpu.SemaphoreType.DMA,
          pltpu.SemaphoreType.DMA,
          pltpu.SemaphoreType.DMA,
          pltpu.SemaphoreType.DMA,
          pltpu.SemaphoreType.DMA,
          pltpu.SemaphoreType.DMA,
          pltpu.SemaphoreType.DMA,
          pltpu.SemaphoreType.DMA,
      ],
  )
  def k(h_hbm, src_hbm, dst_hbm, out_hbm, *refs):
    sset = [refs[3 * i] for i in range(8)]
    dset = [refs[3 * i + 1] for i in range(8)]
    isem = [refs[3 * i + 2] for i in range(8)]
    gbuf = list(refs[24:28])
    zbuf = refs[28]
    acc = refs[29]
    gsem = list(refs[30:34])
    ssem = list(refs[34:38])
    szw = refs[38]

    cid = lax.axis_index("c")
    sid = lax.axis_index("s")
    base_e = sid * _E_PER_SUB

    @pl.loop(0, _ZROWS)
    def _(r):
      @pl.loop(0, D, step=LANES)
      def _(c0):
        zbuf[r, pl.ds(c0, LANES)] = jnp.zeros((LANES,), jnp.float32)

    def zcopy(rc):
      return pltpu.make_async_copy(zbuf, acc.at[pl.ds(rc * _ZROWS, _ZROWS)],
                                   szw)

    @pl.loop(sid, _NROWCH, step=NS)
    def _(rc):
      zcopy(rc).start()

    def i_copies(c, j8):
      soff = cid * _EP + base_e + c * _CH
      doff = base_e + c * _CH
      return (pltpu.make_async_copy(src_hbm.at[pl.ds(soff, _CH)], sset[j8],
                                    isem[j8]),
              pltpu.make_async_copy(dst_hbm.at[pl.ds(doff, _CH)], dset[j8],
                                    isem[j8]))

    def i_start(c, j8):
      for cp in i_copies(c, j8):
        cp.start()

    def i_wait(c, j8):
      for cp in i_copies(c, j8):
        cp.wait()

    def gcopy(j8, b4):
      return pltpu.make_async_copy(h_hbm.at[sset[j8]], gbuf[b4], gsem[b4])

    def scopy(j8, b4):
      return pltpu.make_async_copy(gbuf[b4], acc.at[dset[j8]], ssem[b4])

    def slot(c, j8):
      if isinstance(c, int):
        real = lambda x: 0 <= x < _NECH
      else:
        real = lambda x: True
      cm3, cm4, cp4 = c - 3, c - 4, c + 4
      if real(cm3):
        gcopy((j8 + 5) % 8, (j8 + 1) % 4).wait()
        scopy((j8 + 5) % 8, (j8 + 1) % 4).start(add=True)
      if real(cm4):
        scopy((j8 + 4) % 8, j8 % 4).wait()
      if not isinstance(c, int) or cp4 < _NECH + 2:
        i_start(cp4, (j8 + 4) % 8)
      if real(c):
        i_wait(c, j8)
        gcopy(j8, j8 % 4).start()

    for j in range(4):
      i_start(j, j)
    for cc in range(0, 3):
      slot(cc, cc)

    @pl.loop(sid, _NROWCH, step=NS)
    def _(rc):
      zcopy(rc).wait()

    plsc.subcore_barrier()

    for cc in range(3, 8):
      slot(cc, cc)

    @pl.loop(8, 233, step=8)
    def _(c):
      for j in range(8):
        slot(c + j, j)

    for cc in range(240, 250):
      slot(cc, cc % 8)

    for cc in (247, 248, 249):
      gcopy(cc % 8, cc % 4).wait()
      scopy(cc % 8, cc % 4).start(add=True)
    for cc in (246, 247, 248, 249):
      scopy(cc % 8, cc % 4).wait()
    for cc in (250, 251):
      i_wait(cc, cc % 8)

    plsc.subcore_barrier()

    def wcopy(rc):
      return pltpu.make_async_copy(
          acc.at[pl.ds(rc * _ZROWS, _ZROWS)],
          out_hbm.at[pl.ds(cid * N + rc * _ZROWS, _ZROWS)], szw)

    @pl.loop(sid, _NROWCH, step=NS)
    def _(rc):
      wcopy(rc).start()

    @pl.loop(sid, _NROWCH, step=NS)
    def _(rc):
      wcopy(rc).wait()

  return k(h2, src2, dst_p)


_BN = 1000


def _mlp(h2, agg2, W1, b1, W2, b2):
  M = h2.shape[0]

  def body(h_ref, a_ref, w1_ref, b1_ref, w2_ref, b2_ref, o_ref):
    m = h_ref[...] + a_ref[...]
    z = jnp.dot(m, w1_ref[...], preferred_element_type=jnp.float32)
    z = jnp.maximum(z + b1_ref[...], 0.0)
    o = jnp.dot(z, w2_ref[...], preferred_element_type=jnp.float32)
    o_ref[...] = jnp.maximum(o + b2_ref[...], 0.0)

  return pl.pallas_call(
      body,
      grid=(M // _BN,),
      in_specs=[
          pl.BlockSpec((_BN, D), lambda i: (i, 0)),
          pl.BlockSpec((_BN, D), lambda i: (i, 0)),
          pl.BlockSpec((D, D), lambda i: (0, 0)),
          pl.BlockSpec((1, D), lambda i: (0, 0)),
          pl.BlockSpec((D, D), lambda i: (0, 0)),
          pl.BlockSpec((1, D), lambda i: (0, 0)),
      ],
      out_specs=pl.BlockSpec((_BN, D), lambda i: (i, 0)),
      out_shape=jax.ShapeDtypeStruct((M, D), jnp.float32),
  )(h2, agg2, W1, b1.reshape(1, D), W2, b2.reshape(1, D))


_BNP = 1000
_NBP = N // _BNP


def _pool_tc(h2, batch3):
  def body(b_ref, h_ref, o_ref):
    j = pl.program_id(1)

    @pl.when(j == 0)
    def _():
      o_ref[...] = jnp.zeros_like(o_ref)

    bidx = b_ref[0, 0]
    iota = lax.broadcasted_iota(jnp.int32, (G, _BNP), 0)
    onehot = (iota == bidx[None, :]).astype(jnp.bfloat16)
    h = h_ref[0]
    hi = h.astype(jnp.bfloat16)
    lo = (h - hi.astype(jnp.float32)).astype(jnp.bfloat16)
    o_ref[0] += (jnp.dot(onehot, hi, preferred_element_type=jnp.float32) +
                 jnp.dot(onehot, lo, preferred_element_type=jnp.float32))

  return pl.pallas_call(
      body,
      grid=(2, _NBP),
      in_specs=[
          pl.BlockSpec((1, 1, _BNP), lambda e, j: (j, 0, 0)),
          pl.BlockSpec((1, _BNP, D), lambda e, j: (e, j, 0)),
      ],
      out_specs=pl.BlockSpec((1, G, D), lambda e, j: (e, 0, 0)),
      out_shape=jax.ShapeDtypeStruct((2, G, D), jnp.float32),
  )(batch3, h2.reshape(2, N, D))


def kernel(x, x_s, edge_index, edge_attr, batch, W1_0, b1_0, W2_0, b2_0,
           W1_1, b1_1, W2_1, b2_1, W1_2, b1_2, W2_2, b2_2):
  del edge_attr
  pad = jnp.zeros((_EPAD,), jnp.int32)
  src_p = jnp.concatenate([edge_index[0], pad])
  src2 = jnp.concatenate([src_p, src_p + N])
  dst_p = jnp.concatenate([edge_index[1], pad])
  params = ((W1_0, b1_0, W2_0, b2_0),
            (W1_1, b1_1, W2_1, b2_1),
            (W1_2, b1_2, W2_2, b2_2))

  batch3 = batch.reshape(_NBP, 1, _BNP)
  h = jnp.concatenate([x, x_s], axis=0)
  hs, pools = [], []
  for (W1, b1, W2, b2) in params:
    agg = _seg_sum_stacked(h, src2, dst_p)
    h = _mlp(h, agg, W1, b1, W2, b2)
    hs.append(h)
    pools.append(_pool_tc(h, batch3))

  n_x = jnp.concatenate([p[0] for p in pools], axis=1)
  n_xs = jnp.concatenate([p[1] for p in pools], axis=1)
  g_x = jnp.concatenate([hs[0][:N], hs[1][:N], hs[2][:N]], axis=1)
  g_xs = jnp.concatenate([hs[0][N:], hs[1][N:], hs[2][N:]], axis=1)
  return (n_x, g_x, n_xs, g_xs)


# speedup vs baseline: 8.0764x; 1.0287x over previous
import functools

import jax
import jax.numpy as jnp
from jax import lax
from jax.experimental import pallas as pl
from jax.experimental.pallas import tpu as pltpu
from jax.experimental.pallas import tpu_sc as plsc

N = 10000
E = 320000
D = 128
G = 512
NC = 2
NS = 16
LANES = 16

_MESH = plsc.VectorSubcoreMesh(core_axis_name="c", subcore_axis_name="s")


_E_PER_SUB = E // NS
_CH = 80
_NECH = _E_PER_SUB // _CH
_EPAD = 2 * _CH
_EP = E + _EPAD
_ZROWS = 40
_NROWCH = N // _ZROWS


def _seg_sum_stacked(h2, src2, dst_p):
  idx_scr = []
  for _ in range(8):
    idx_scr += [pltpu.VMEM((_CH,), jnp.int32),
                pltpu.VMEM((_CH,), jnp.int32),
                pltpu.SemaphoreType.DMA]

  @functools.partial(
      pl.kernel,
      mesh=_MESH,
      out_type=jax.ShapeDtypeStruct((2 * N, D), jnp.float32),
      scratch_types=idx_scr + [
          pltpu.VMEM((_CH, D), jnp.float32),
          pltpu.VMEM((_CH, D), jnp.float32),
          pltpu.VMEM((_CH, D), jnp.float32),
          pltpu.VMEM((_CH, D), jnp.float32),
          pltpu.VMEM((_ZROWS, D), jnp.float32),
          pltpu.VMEM_SHARED((N, D), jnp.float32),
          pltpu.SemaphoreType.DMA,
          pltpu.SemaphoreType.DMA,
          pltpu.SemaphoreType.DMA,
          pltpu.SemaphoreType.DMA,
          pltpu.SemaphoreType.DMA,
          pltpu.SemaphoreType.DMA,
          pltpu.SemaphoreType.DMA,
          pltpu.SemaphoreType.DMA,
          pltpu.SemaphoreType.DMA,
      ],
  )
  def k(h_hbm, src_hbm, dst_hbm, out_hbm, *refs):
    sset = [refs[3 * i] for i in range(8)]
    dset = [refs[3 * i + 1] for i in range(8)]
    isem = [refs[3 * i + 2] for i in range(8)]
    gbuf = list(refs[24:28])
    zbuf = refs[28]
    acc = refs[29]
    gsem = list(refs[30:34])
    ssem = list(refs[34:38])
    szw = refs[38]

    cid = lax.axis_index("c")
    sid = lax.axis_index("s")
    base_e = sid * _E_PER_SUB

    @pl.loop(0, _ZROWS)
    def _(r):
      @pl.loop(0, D, step=LANES)
      def _(c0):
        zbuf[r, pl.ds(c0, LANES)] = jnp.zeros((LANES,), jnp.float32)

    def zcopy(rc):
      return pltpu.make_async_copy(zbuf, acc.at[pl.ds(rc * _ZROWS, _ZROWS)],
                                   szw)

    @pl.loop(sid, _NROWCH, step=NS)
    def _(rc):
      zcopy(rc).start()

    def i_copies(c, j8):
      soff = cid * _EP + base_e + c * _CH
      doff = base_e + c * _CH
      return (pltpu.make_async_copy(src_hbm.at[pl.ds(soff, _CH)], sset[j8],
                                    isem[j8]),
              pltpu.make_async_copy(dst_hbm.at[pl.ds(doff, _CH)], dset[j8],
                                    isem[j8]))

    def i_start(c, j8):
      for cp in i_copies(c, j8):
        cp.start()

    def i_wait(c, j8):
      for cp in i_copies(c, j8):
        cp.wait()

    def gcopy(j8, b4):
      return pltpu.make_async_copy(h_hbm.at[sset[j8]], gbuf[b4], gsem[b4])

    def scopy(j8, b4):
      return pltpu.make_async_copy(gbuf[b4], acc.at[dset[j8]], ssem[b4])

    def slot(c, j8):
      if isinstance(c, int):
        real = lambda x: 0 <= x < _NECH
      else:
        real = lambda x: True
      cm3, cm4, cp4 = c - 3, c - 4, c + 4
      if real(cm3):
        gcopy((j8 + 5) % 8, (j8 + 1) % 4).wait()
        scopy((j8 + 5) % 8, (j8 + 1) % 4).start(add=True)
      if real(cm4):
        scopy((j8 + 4) % 8, j8 % 4).wait()
      if not isinstance(c, int) or cp4 < _NECH + 2:
        i_start(cp4, (j8 + 4) % 8)
      if real(c):
        i_wait(c, j8)
        gcopy(j8, j8 % 4).start()

    for j in range(4):
      i_start(j, j)
    for cc in range(0, 3):
      slot(cc, cc)

    @pl.loop(sid, _NROWCH, step=NS)
    def _(rc):
      zcopy(rc).wait()

    plsc.subcore_barrier()

    for cc in range(3, 8):
      slot(cc, cc)

    @pl.loop(8, 233, step=8)
    def _(c):
      for j in range(8):
        slot(c + j, j)

    for cc in range(240, 250):
      slot(cc, cc % 8)

    for cc in (247, 248, 249):
      gcopy(cc % 8, cc % 4).wait()
      scopy(cc % 8, cc % 4).start(add=True)
    for cc in (246, 247, 248, 249):
      scopy(cc % 8, cc % 4).wait()
    for cc in (250, 251):
      i_wait(cc, cc % 8)

    plsc.subcore_barrier()

    def wcopy(rc):
      return pltpu.make_async_copy(
          acc.at[pl.ds(rc * _ZROWS, _ZROWS)],
          out_hbm.at[pl.ds(cid * N + rc * _ZROWS, _ZROWS)], szw)

    @pl.loop(sid, _NROWCH, step=NS)
    def _(rc):
      wcopy(rc).start()

    @pl.loop(sid, _NROWCH, step=NS)
    def _(rc):
      wcopy(rc).wait()

  return k(h2, src2, dst_p)


_BN = 2000


def _mlp(h2, agg2, W1, b1, W2, b2):
  M = h2.shape[0]

  def body(h_ref, a_ref, w1_ref, b1_ref, w2_ref, b2_ref, o_ref):
    m = h_ref[...] + a_ref[...]
    z = jnp.dot(m, w1_ref[...], preferred_element_type=jnp.float32)
    z = jnp.maximum(z + b1_ref[...], 0.0)
    o = jnp.dot(z, w2_ref[...], preferred_element_type=jnp.float32)
    o_ref[...] = jnp.maximum(o + b2_ref[...], 0.0)

  return pl.pallas_call(
      body,
      grid=(M // _BN,),
      in_specs=[
          pl.BlockSpec((_BN, D), lambda i: (i, 0)),
          pl.BlockSpec((_BN, D), lambda i: (i, 0)),
          pl.BlockSpec((D, D), lambda i: (0, 0)),
          pl.BlockSpec((1, D), lambda i: (0, 0)),
          pl.BlockSpec((D, D), lambda i: (0, 0)),
          pl.BlockSpec((1, D), lambda i: (0, 0)),
      ],
      out_specs=pl.BlockSpec((_BN, D), lambda i: (i, 0)),
      out_shape=jax.ShapeDtypeStruct((M, D), jnp.float32),
  )(h2, agg2, W1, b1.reshape(1, D), W2, b2.reshape(1, D))


_BNP = 1000
_NBP = N // _BNP


def _pool_tc(h2, batch3):
  def body(b_ref, h_ref, o_ref):
    j = pl.program_id(1)

    @pl.when(j == 0)
    def _():
      o_ref[...] = jnp.zeros_like(o_ref)

    bidx = b_ref[0, 0]
    iota = lax.broadcasted_iota(jnp.int32, (G, _BNP), 0)
    onehot = (iota == bidx[None, :]).astype(jnp.bfloat16)
    h = h_ref[0]
    hi = h.astype(jnp.bfloat16)
    lo = (h - hi.astype(jnp.float32)).astype(jnp.bfloat16)
    o_ref[0] += (jnp.dot(onehot, hi, preferred_element_type=jnp.float32) +
                 jnp.dot(onehot, lo, preferred_element_type=jnp.float32))

  return pl.pallas_call(
      body,
      grid=(2, _NBP),
      in_specs=[
          pl.BlockSpec((1, 1, _BNP), lambda e, j: (j, 0, 0)),
          pl.BlockSpec((1, _BNP, D), lambda e, j: (e, j, 0)),
      ],
      out_specs=pl.BlockSpec((1, G, D), lambda e, j: (e, 0, 0)),
      out_shape=jax.ShapeDtypeStruct((2, G, D), jnp.float32),
  )(batch3, h2.reshape(2, N, D))


def kernel(x, x_s, edge_index, edge_attr, batch, W1_0, b1_0, W2_0, b2_0,
           W1_1, b1_1, W2_1, b2_1, W1_2, b1_2, W2_2, b2_2):
  del edge_attr
  pad = jnp.zeros((_EPAD,), jnp.int32)
  src_p = jnp.concatenate([edge_index[0], pad])
  src2 = jnp.concatenate([src_p, src_p + N])
  dst_p = jnp.concatenate([edge_index[1], pad])
  params = ((W1_0, b1_0, W2_0, b2_0),
            (W1_1, b1_1, W2_1, b2_1),
            (W1_2, b1_2, W2_2, b2_2))

  batch3 = batch.reshape(_NBP, 1, _BNP)
  h = jnp.concatenate([x, x_s], axis=0)
  hs, pools = [], []
  for (W1, b1, W2, b2) in params:
    agg = _seg_sum_stacked(h, src2, dst_p)
    h = _mlp(h, agg, W1, b1, W2, b2)
    hs.append(h)
    pools.append(_pool_tc(h, batch3))

  n_x = jnp.concatenate([p[0] for p in pools], axis=1)
  n_xs = jnp.concatenate([p[1] for p in pools], axis=1)
  g_x = jnp.concatenate([hs[0][:N], hs[1][:N], hs[2][:N]], axis=1)
  g_xs = jnp.concatenate([hs[0][N:], hs[1][N:], hs[2][N:]], axis=1)
  return (n_x, g_x, n_xs, g_xs)
